# baseline (speedup 1.0000x reference)
import functools

import jax
import jax.numpy as jnp
from jax import lax
from jax.experimental import pallas as pl
from jax.experimental.pallas import tpu as pltpu
from jax.experimental.pallas import tpu_sc as plsc

N = 10000
E = 320000
D = 128
H = 128
C = 40
G = 64

NSC = 2
NTILE = 16
NW = NSC * NTILE
CHUNK = 128
BLK = 128
NBLK = 79
NROWP = NBLK * BLK
RPT = NROWP // NTILE
NCHA = 96
NCHB = 64
NCH_MAX = NCHA
PCH = NCH_MAX // 2
EPAD = NTILE * (NCHA + NCHB) * CHUNK
NEA = NTILE * NCHA * CHUNK
TRASH = N

_mesh = plsc.VectorSubcoreMesh(core_axis_name="c", subcore_axis_name="s")


def _edge_scatter_kernel(width):
    @functools.partial(
        pl.kernel,
        out_type=jax.ShapeDtypeStruct((NSC, NROWP, width), jnp.float32),
        mesh=_mesh,
        scratch_types=[
            pltpu.VMEM((NCH_MAX, CHUNK), jnp.int32),
            pltpu.VMEM((NCH_MAX, CHUNK), jnp.int32),
            pltpu.VMEM((CHUNK, width), jnp.float32),
            pltpu.VMEM_SHARED((NROWP, width), jnp.float32),
            pltpu.SemaphoreType.DMA,
        ],
    )
    def k(g_hbm, gi_hbm, si_hbm, z_hbm, out_hbm, gi_v, si_v, rows_v, acc_sh, sem):
        c = lax.axis_index("c")
        s = lax.axis_index("s")
        wid = c * NTILE + s
        base = s * RPT
        nch = lax.select(c == 0, NCHA, NCHB)
        pltpu.sync_copy(z_hbm, rows_v)
        for kk in range(4):
            pltpu.sync_copy(rows_v, acc_sh.at[pl.ds(base + kk * CHUNK, CHUNK)])
        pltpu.sync_copy(rows_v.at[pl.ds(0, RPT - 4 * CHUNK)],
                        acc_sh.at[pl.ds(base + 4 * CHUNK, RPT - 4 * CHUNK)])
        pltpu.sync_copy(gi_hbm.at[wid], gi_v)
        pltpu.sync_copy(si_hbm.at[wid], si_v)
        plsc.subcore_barrier()

        def body(ch, carry):
            pltpu.async_copy(g_hbm.at[gi_v.at[ch]], rows_v, sem).wait()
            pltpu.sync_copy(rows_v, acc_sh.at[si_v.at[ch]], add=True)
            return carry

        lax.fori_loop(0, nch, body, 0)
        plsc.subcore_barrier()
        for kk in range(4):
            pltpu.sync_copy(acc_sh.at[pl.ds(base + kk * CHUNK, CHUNK)], rows_v)
            pltpu.sync_copy(rows_v, out_hbm.at[c, pl.ds(base + kk * CHUNK, CHUNK)])
        tail = RPT - 4 * CHUNK
        pltpu.sync_copy(acc_sh.at[pl.ds(base + 4 * CHUNK, tail)],
                        rows_v.at[pl.ds(0, tail)])
        pltpu.sync_copy(rows_v.at[pl.ds(0, tail)],
                        out_hbm.at[c, pl.ds(base + 4 * CHUNK, tail)])

    return k


_edge_scatter_h = _edge_scatter_kernel(H)


@functools.partial(
    pl.kernel,
    out_type=jax.ShapeDtypeStruct((NSC * NROWP,), jnp.float32),
    mesh=_mesh,
    scratch_types=[
        pltpu.VMEM((NCH_MAX, CHUNK), jnp.int32),
        pltpu.VMEM((CHUNK,), jnp.float32),
        pltpu.VMEM((RPT,), jnp.float32),
        pltpu.VMEM_SHARED((NROWP,), jnp.float32),
        pltpu.SemaphoreType.DMA,
    ],
)
def _degree_kernel(si_hbm, ones_hbm, z_hbm, out_hbm, si_v, ones_v, zb_v, acc_sh, sem):
    c = lax.axis_index("c")
    s = lax.axis_index("s")
    wid = c * NTILE + s
    base = s * RPT
    nch = lax.select(c == 0, NCHA, NCHB)
    pltpu.sync_copy(z_hbm, zb_v)
    pltpu.sync_copy(zb_v, acc_sh.at[pl.ds(base, RPT)])
    pltpu.sync_copy(ones_hbm, ones_v)
    pltpu.sync_copy(si_hbm.at[wid], si_v)
    plsc.subcore_barrier()

    def body(ch, carry):
        pltpu.sync_copy(ones_v, acc_sh.at[si_v.at[ch]], add=True)
        return carry

    lax.fori_loop(0, nch, body, 0)
    plsc.subcore_barrier()
    pltpu.sync_copy(acc_sh.at[pl.ds(base, RPT)], zb_v)
    pltpu.sync_copy(zb_v, out_hbm.at[pl.ds(c * NROWP + base, RPT)])


def _dis_body(deg_ref, out_ref):
    d = deg_ref[...]
    out_ref[...] = lax.rsqrt(d[0] + d[1] + 1.0)


_dis_call = pl.pallas_call(
    _dis_body,
    out_shape=jax.ShapeDtypeStruct((NBLK, BLK), jnp.float32),
)


def _prep_body(x_ref, dis_ref, w_ref, o_ref):
    d = dis_ref[0, 0, :]
    o_ref[...] = (x_ref[...] @ w_ref[...]) * d[:, None]


_prep_call = pl.pallas_call(
    _prep_body,
    grid=(NBLK,),
    in_specs=[
        pl.BlockSpec((BLK, D), lambda i: (i, 0)),
        pl.BlockSpec((1, 1, BLK), lambda i: (i, 0, 0)),
        pl.BlockSpec((D, H), lambda i: (0, 0)),
    ],
    out_specs=pl.BlockSpec((BLK, H), lambda i: (i, 0)),
    out_shape=jax.ShapeDtypeStruct((NROWP, H), jnp.float32),
)


def _layer_body(s_ref, g_ref, dis_ref, w_ref, b_ref, gn_ref):
    d = dis_ref[0, 0, :]
    sp = s_ref[...]
    h = jnp.maximum((sp[0] + sp[1] + g_ref[...]) * d[:, None] + b_ref[...], 0.0)
    gn_ref[...] = (h @ w_ref[...]) * d[:, None]


_layer_call = pl.pallas_call(
    _layer_body,
    grid=(NBLK,),
    in_specs=[
        pl.BlockSpec((NSC, BLK, H), lambda i: (0, i, 0)),
        pl.BlockSpec((BLK, H), lambda i: (i, 0)),
        pl.BlockSpec((1, 1, BLK), lambda i: (i, 0, 0)),
        pl.BlockSpec((H, H), lambda i: (0, 0)),
        pl.BlockSpec((1, H), lambda i: (0, 0)),
    ],
    out_specs=pl.BlockSpec((BLK, H), lambda i: (i, 0)),
    out_shape=jax.ShapeDtypeStruct((NROWP, H), jnp.float32),
)


def _comb5_body(s_ref, g_ref, dis_ref, b_ref, h_ref):
    d = dis_ref[0, 0, :]
    sp = s_ref[...]
    h_ref[...] = jnp.maximum((sp[0] + sp[1] + g_ref[...]) * d[:, None] + b_ref[...], 0.0)


_comb5_call = pl.pallas_call(
    _comb5_body,
    grid=(NBLK,),
    in_specs=[
        pl.BlockSpec((NSC, BLK, H), lambda i: (0, i, 0)),
        pl.BlockSpec((BLK, H), lambda i: (i, 0)),
        pl.BlockSpec((1, 1, BLK), lambda i: (i, 0, 0)),
        pl.BlockSpec((1, H), lambda i: (0, 0)),
    ],
    out_specs=pl.BlockSpec((BLK, H), lambda i: (i, 0)),
    out_shape=jax.ShapeDtypeStruct((NROWP, H), jnp.float32),
)


def _cnt_body(b_ref, cnt_ref):
    @pl.when(pl.program_id(0) == 0)
    def _init():
        cnt_ref[...] = jnp.zeros_like(cnt_ref)

    bb = b_ref[0, 0, :]
    oh = (bb[:, None] == lax.broadcasted_iota(jnp.int32, (BLK, G), 1)).astype(jnp.float32)
    cnt_ref[...] += jnp.sum(oh, axis=0, keepdims=True)


_cnt_call = pl.pallas_call(
    _cnt_body,
    grid=(NBLK,),
    in_specs=[pl.BlockSpec((1, 1, BLK), lambda i: (i, 0, 0))],
    out_specs=pl.BlockSpec((1, G), lambda i: (0, 0)),
    out_shape=jax.ShapeDtypeStruct((1, G), jnp.float32),
)


def _u_body(b_ref, dis_ref, cnt_ref, u_ref):
    bb = b_ref[0, 0, :]
    d = dis_ref[0, 0, :]
    oh = (bb[:, None] == lax.broadcasted_iota(jnp.int32, (BLK, H), 1)).astype(jnp.float32)
    cinv = 1.0 / jnp.maximum(cnt_ref[...], 1.0)
    cinv_p = jnp.concatenate([cinv, jnp.ones((1, H - G), jnp.float32)], axis=1)
    u_ref[...] = oh * d[:, None] * cinv_p


_u_call = pl.pallas_call(
    _u_body,
    grid=(NBLK,),
    in_specs=[
        pl.BlockSpec((1, 1, BLK), lambda i: (i, 0, 0)),
        pl.BlockSpec((1, 1, BLK), lambda i: (i, 0, 0)),
        pl.BlockSpec((1, G), lambda i: (0, 0)),
    ],
    out_specs=pl.BlockSpec((BLK, H), lambda i: (i, 0)),
    out_shape=jax.ShapeDtypeStruct((NROWP, H), jnp.float32),
)


def _qt_body(t_ref, u_ref, dis_ref, qt_ref):
    d = dis_ref[0, 0, :]
    tp = t_ref[...]
    rid = pl.program_id(0) * BLK + lax.broadcasted_iota(jnp.int32, (BLK, G), 0)
    qt = ((tp[0] + tp[1] + u_ref[...]) * d[:, None])[:, :G]
    qt_ref[...] = jnp.where(rid < N, qt, 0.0)


_qt_call = pl.pallas_call(
    _qt_body,
    grid=(NBLK,),
    in_specs=[
        pl.BlockSpec((NSC, BLK, H), lambda i: (0, i, 0)),
        pl.BlockSpec((BLK, H), lambda i: (i, 0)),
        pl.BlockSpec((1, 1, BLK), lambda i: (i, 0, 0)),
    ],
    out_specs=pl.BlockSpec((BLK, G), lambda i: (i, 0)),
    out_shape=jax.ShapeDtypeStruct((NROWP, G), jnp.float32),
)


def _final_body(qt_ref, h_ref, w6_ref, wlin_ref, b6_ref, blin_ref, o_ref):
    pooled = lax.dot_general(qt_ref[...], h_ref[...], (((0,), (0,)), ((), ())),
                             preferred_element_type=jnp.float32)
    wc = w6_ref[...] @ wlin_ref[...]
    o_ref[...] = pooled @ wc + b6_ref[...] @ wlin_ref[...] + blin_ref[...]


_final_call = pl.pallas_call(
    _final_body,
    out_shape=jax.ShapeDtypeStruct((G, C), jnp.float32),
)


def kernel(x, edge_index, batch, W1, b1, W2, b2, W3, b3, W4, b4, W5, b5, W6, b6,
           Wlin, blin):
    f32 = jnp.float32
    src = edge_index[0]
    dst = edge_index[1]
    pad = EPAD - E

    def slab(idx, fill):
        idx_p = jnp.concatenate([idx, jnp.full((pad,), fill, jnp.int32)])
        part_a = idx_p[:NEA].reshape(NTILE, NCHA, CHUNK)
        part_a = jnp.pad(part_a, ((0, 0), (0, NCH_MAX - NCHA), (0, 0)),
                         constant_values=fill)
        part_b = idx_p[NEA:].reshape(NTILE, NCHB, CHUNK)
        part_b = jnp.pad(part_b, ((0, 0), (0, NCH_MAX - NCHB), (0, 0)),
                         constant_values=fill)
        return jnp.concatenate([part_a, part_b], axis=0)

    fwd_gi = slab(src, 0)
    fwd_si = slab(dst, TRASH)
    rev_gi = slab(dst, 0)
    rev_si = slab(src, TRASH)

    zeros_h = jnp.zeros((CHUNK, H), f32)
    zeros_1 = jnp.zeros((RPT,), f32)
    ones_1 = jnp.ones((CHUNK,), f32)

    x_p = jnp.pad(x, ((0, NROWP - N), (0, 0)))
    batch3 = jnp.concatenate(
        [batch, jnp.full((NROWP - N,), G, jnp.int32)]).reshape(NBLK, 1, BLK)
    b1r = b1.reshape(1, H); b2r = b2.reshape(1, H); b3r = b3.reshape(1, H)
    b4r = b4.reshape(1, H); b5r = b5.reshape(1, H); b6r = b6.reshape(1, H)
    blinr = blin.reshape(1, C)

    deg = _degree_kernel(fwd_si, ones_1, zeros_1)
    dis3 = _dis_call(deg.reshape(NSC, NBLK, BLK)).reshape(NBLK, 1, BLK)

    cnt = _cnt_call(batch3)
    u = _u_call(batch3, dis3, cnt)
    t = _edge_scatter_h(u, rev_gi, rev_si, zeros_h)
    qt = _qt_call(t, u, dis3)

    g = _prep_call(x_p, dis3, W1)
    Ws = [W2, W3, W4, W5]
    bs = [b1r, b2r, b3r, b4r]
    for l in range(4):
        s = _edge_scatter_h(g, fwd_gi, fwd_si, zeros_h)
        g = _layer_call(s, g, dis3, Ws[l], bs[l])
    s = _edge_scatter_h(g, fwd_gi, fwd_si, zeros_h)
    h5 = _comb5_call(s, g, dis3, b5r)

    return _final_call(qt, h5, W6, Wlin, b6r, blinr)


# speedup vs baseline: 2.0155x; 2.0155x over previous
import functools

import jax
import jax.numpy as jnp
from jax import lax
from jax.experimental import pallas as pl
from jax.experimental.pallas import tpu as pltpu
from jax.experimental.pallas import tpu_sc as plsc

N = 10000
E = 320000
D = 128
H = 128
C = 40
G = 64

NSC = 2
NTILE = 16
NW = NSC * NTILE
CHUNK = 128
BLK = 128
NBLK = 79
NROWP = NBLK * BLK
RPT = NROWP // NTILE
NCHA = 95
NCHB = 62
NCH_MAX = NCHA
PCH = NCH_MAX // 2
EPAD = NTILE * (NCHA + NCHB) * CHUNK
NEA = NTILE * NCHA * CHUNK
TRASH = N

_mesh = plsc.VectorSubcoreMesh(core_axis_name="c", subcore_axis_name="s")


def _edge_scatter_kernel(width):
    @functools.partial(
        pl.kernel,
        out_type=jax.ShapeDtypeStruct((NSC, NROWP, width), jnp.float32),
        mesh=_mesh,
        scratch_types=[
            pltpu.VMEM((NCH_MAX, CHUNK), jnp.int32),
            pltpu.VMEM((NCH_MAX, CHUNK), jnp.int32),
            pltpu.VMEM((CHUNK, width), jnp.float32),
            pltpu.VMEM_SHARED((NROWP, width), jnp.float32),
            pltpu.SemaphoreType.DMA,
        ],
    )
    def k(g_hbm, gi_hbm, si_hbm, z_hbm, out_hbm, gi_v, si_v, rows_v, acc_sh, sem):
        c = lax.axis_index("c")
        s = lax.axis_index("s")
        wid = c * NTILE + s
        base = s * RPT
        nch = lax.select(c == 0, NCHA, NCHB)
        pltpu.sync_copy(z_hbm, rows_v)
        for kk in range(4):
            pltpu.sync_copy(rows_v, acc_sh.at[pl.ds(base + kk * CHUNK, CHUNK)])
        pltpu.sync_copy(rows_v.at[pl.ds(0, RPT - 4 * CHUNK)],
                        acc_sh.at[pl.ds(base + 4 * CHUNK, RPT - 4 * CHUNK)])
        pltpu.sync_copy(gi_hbm.at[wid], gi_v)
        pltpu.sync_copy(si_hbm.at[wid], si_v)
        plsc.subcore_barrier()

        def body(ch, carry):
            pltpu.async_copy(g_hbm.at[gi_v.at[ch]], rows_v, sem).wait()
            pltpu.sync_copy(rows_v, acc_sh.at[si_v.at[ch]], add=True)
            return carry

        lax.fori_loop(0, nch, body, 0)
        plsc.subcore_barrier()
        for kk in range(4):
            pltpu.sync_copy(acc_sh.at[pl.ds(base + kk * CHUNK, CHUNK)], rows_v)
            pltpu.sync_copy(rows_v, out_hbm.at[c, pl.ds(base + kk * CHUNK, CHUNK)])
        tail = RPT - 4 * CHUNK
        pltpu.sync_copy(acc_sh.at[pl.ds(base + 4 * CHUNK, tail)],
                        rows_v.at[pl.ds(0, tail)])
        pltpu.sync_copy(rows_v.at[pl.ds(0, tail)],
                        out_hbm.at[c, pl.ds(base + 4 * CHUNK, tail)])

    return k


_edge_scatter_h = _edge_scatter_kernel(H)


@functools.partial(
    pl.kernel,
    out_type=jax.ShapeDtypeStruct((NSC * NROWP,), jnp.float32),
    mesh=_mesh,
    scratch_types=[
        pltpu.VMEM((NCH_MAX, CHUNK), jnp.int32),
        pltpu.VMEM((CHUNK,), jnp.float32),
        pltpu.VMEM((RPT,), jnp.float32),
        pltpu.VMEM_SHARED((NROWP,), jnp.float32),
        pltpu.SemaphoreType.DMA,
    ],
)
def _degree_kernel(si_hbm, ones_hbm, z_hbm, out_hbm, si_v, ones_v, zb_v, acc_sh, sem):
    c = lax.axis_index("c")
    s = lax.axis_index("s")
    wid = c * NTILE + s
    base = s * RPT
    nch = lax.select(c == 0, NCHA, NCHB)
    pltpu.sync_copy(z_hbm, zb_v)
    pltpu.sync_copy(zb_v, acc_sh.at[pl.ds(base, RPT)])
    pltpu.sync_copy(ones_hbm, ones_v)
    pltpu.sync_copy(si_hbm.at[wid], si_v)
    plsc.subcore_barrier()

    def body(ch, carry):
        pltpu.sync_copy(ones_v, acc_sh.at[si_v.at[ch]], add=True)
        return carry

    lax.fori_loop(0, nch, body, 0)
    plsc.subcore_barrier()
    pltpu.sync_copy(acc_sh.at[pl.ds(base, RPT)], zb_v)
    pltpu.sync_copy(zb_v, out_hbm.at[pl.ds(c * NROWP + base, RPT)])


def _dis_body(deg_ref, out_ref):
    d = deg_ref[...]
    out_ref[...] = lax.rsqrt(d[0] + d[1] + 1.0)


_dis_call = pl.pallas_call(
    _dis_body,
    out_shape=jax.ShapeDtypeStruct((NBLK, BLK), jnp.float32),
)


def _prep_body(x_ref, dis_ref, w_ref, o_ref):
    d = dis_ref[0, 0, :]
    o_ref[...] = (x_ref[...] @ w_ref[...]) * d[:, None]


_prep_call = pl.pallas_call(
    _prep_body,
    grid=(NBLK,),
    in_specs=[
        pl.BlockSpec((BLK, D), lambda i: (i, 0)),
        pl.BlockSpec((1, 1, BLK), lambda i: (i, 0, 0)),
        pl.BlockSpec((D, H), lambda i: (0, 0)),
    ],
    out_specs=pl.BlockSpec((BLK, H), lambda i: (i, 0)),
    out_shape=jax.ShapeDtypeStruct((NROWP, H), jnp.float32),
)


def _layer_body(s_ref, g_ref, dis_ref, w_ref, b_ref, gn_ref, h_ref):
    d = dis_ref[0, 0, :]
    sp = s_ref[...]
    h = jnp.maximum((sp[0] + sp[1] + g_ref[...]) * d[:, None] + b_ref[...], 0.0)
    h_ref[...] = h
    gn_ref[...] = (h @ w_ref[...]) * d[:, None]


_layer_call = pl.pallas_call(
    _layer_body,
    grid=(NBLK,),
    in_specs=[
        pl.BlockSpec((NSC, BLK, H), lambda i: (0, i, 0)),
        pl.BlockSpec((BLK, H), lambda i: (i, 0)),
        pl.BlockSpec((1, 1, BLK), lambda i: (i, 0, 0)),
        pl.BlockSpec((H, H), lambda i: (0, 0)),
        pl.BlockSpec((1, H), lambda i: (0, 0)),
    ],
    out_specs=[
        pl.BlockSpec((BLK, H), lambda i: (i, 0)),
        pl.BlockSpec((BLK, H), lambda i: (i, 0)),
    ],
    out_shape=[
        jax.ShapeDtypeStruct((NROWP, H), jnp.float32),
        jax.ShapeDtypeStruct((NROWP, H), jnp.float32),
    ],
)


def _comb5_body(s_ref, g_ref, dis_ref, b_ref, h_ref):
    d = dis_ref[0, 0, :]
    sp = s_ref[...]
    h_ref[...] = jnp.maximum((sp[0] + sp[1] + g_ref[...]) * d[:, None] + b_ref[...], 0.0)


_comb5_call = pl.pallas_call(
    _comb5_body,
    grid=(NBLK,),
    in_specs=[
        pl.BlockSpec((NSC, BLK, H), lambda i: (0, i, 0)),
        pl.BlockSpec((BLK, H), lambda i: (i, 0)),
        pl.BlockSpec((1, 1, BLK), lambda i: (i, 0, 0)),
        pl.BlockSpec((1, H), lambda i: (0, 0)),
    ],
    out_specs=pl.BlockSpec((BLK, H), lambda i: (i, 0)),
    out_shape=jax.ShapeDtypeStruct((NROWP, H), jnp.float32),
)


def _cnt_body(b_ref, cnt_ref):
    @pl.when(pl.program_id(0) == 0)
    def _init():
        cnt_ref[...] = jnp.zeros_like(cnt_ref)

    bb = b_ref[0, 0, :]
    oh = (bb[:, None] == lax.broadcasted_iota(jnp.int32, (BLK, G), 1)).astype(jnp.float32)
    cnt_ref[...] += jnp.sum(oh, axis=0, keepdims=True)


_cnt_call = pl.pallas_call(
    _cnt_body,
    grid=(NBLK,),
    in_specs=[pl.BlockSpec((1, 1, BLK), lambda i: (i, 0, 0))],
    out_specs=pl.BlockSpec((1, G), lambda i: (0, 0)),
    out_shape=jax.ShapeDtypeStruct((1, G), jnp.float32),
)


def _u_body(b_ref, dis_ref, cnt_ref, u_ref):
    bb = b_ref[0, 0, :]
    d = dis_ref[0, 0, :]
    oh = (bb[:, None] == lax.broadcasted_iota(jnp.int32, (BLK, H), 1)).astype(jnp.float32)
    cinv = 1.0 / jnp.maximum(cnt_ref[...], 1.0)
    cinv_p = jnp.concatenate([cinv, jnp.ones((1, H - G), jnp.float32)], axis=1)
    u_ref[...] = oh * d[:, None] * cinv_p


_u_call = pl.pallas_call(
    _u_body,
    grid=(NBLK,),
    in_specs=[
        pl.BlockSpec((1, 1, BLK), lambda i: (i, 0, 0)),
        pl.BlockSpec((1, 1, BLK), lambda i: (i, 0, 0)),
        pl.BlockSpec((1, G), lambda i: (0, 0)),
    ],
    out_specs=pl.BlockSpec((BLK, H), lambda i: (i, 0)),
    out_shape=jax.ShapeDtypeStruct((NROWP, H), jnp.float32),
)


def _qt_body(t_ref, u_ref, dis_ref, qt_ref):
    d = dis_ref[0, 0, :]
    tp = t_ref[...]
    rid = pl.program_id(0) * BLK + lax.broadcasted_iota(jnp.int32, (BLK, G), 0)
    qt = ((tp[0] + tp[1] + u_ref[...]) * d[:, None])[:, :G]
    qt_ref[...] = jnp.where(rid < N, qt, 0.0)


_qt_call = pl.pallas_call(
    _qt_body,
    grid=(NBLK,),
    in_specs=[
        pl.BlockSpec((NSC, BLK, H), lambda i: (0, i, 0)),
        pl.BlockSpec((BLK, H), lambda i: (i, 0)),
        pl.BlockSpec((1, 1, BLK), lambda i: (i, 0, 0)),
    ],
    out_specs=pl.BlockSpec((BLK, G), lambda i: (i, 0)),
    out_shape=jax.ShapeDtypeStruct((NROWP, G), jnp.float32),
)


def _final_body(qt_ref, h_ref, w6_ref, wlin_ref, b6_ref, blin_ref, o_ref):
    pooled = lax.dot_general(qt_ref[...], h_ref[...], (((0,), (0,)), ((), ())),
                             preferred_element_type=jnp.float32)
    wc = w6_ref[...] @ wlin_ref[...]
    o_ref[...] = pooled @ wc + b6_ref[...] @ wlin_ref[...] + blin_ref[...]


_final_call = pl.pallas_call(
    _final_body,
    out_shape=jax.ShapeDtypeStruct((G, C), jnp.float32),
)


def kernel(x, edge_index, batch, W1, b1, W2, b2, W3, b3, W4, b4, W5, b5, W6, b6,
           Wlin, blin):
    f32 = jnp.float32
    src = edge_index[0]
    dst = edge_index[1]
    pad = EPAD - E

    def slab(idx, fill):
        idx_p = jnp.concatenate([idx, jnp.full((pad,), fill, jnp.int32)])
        part_a = idx_p[:NEA].reshape(NTILE, NCHA, CHUNK)
        part_a = jnp.pad(part_a, ((0, 0), (0, NCH_MAX - NCHA), (0, 0)),
                         constant_values=fill)
        part_b = idx_p[NEA:].reshape(NTILE, NCHB, CHUNK)
        part_b = jnp.pad(part_b, ((0, 0), (0, NCH_MAX - NCHB), (0, 0)),
                         constant_values=fill)
        return jnp.concatenate([part_a, part_b], axis=0)

    fwd_gi = slab(src, 0)
    fwd_si = slab(dst, TRASH)
    rev_gi = slab(dst, 0)
    rev_si = slab(src, TRASH)

    zeros_h = jnp.zeros((CHUNK, H), f32)
    zeros_1 = jnp.zeros((RPT,), f32)
    ones_1 = jnp.ones((CHUNK,), f32)

    x_p = jnp.pad(x, ((0, NROWP - N), (0, 0)))
    batch3 = jnp.concatenate(
        [batch, jnp.full((NROWP - N,), G, jnp.int32)]).reshape(NBLK, 1, BLK)
    b1r = b1.reshape(1, H); b2r = b2.reshape(1, H); b3r = b3.reshape(1, H)
    b4r = b4.reshape(1, H); b5r = b5.reshape(1, H); b6r = b6.reshape(1, H)
    blinr = blin.reshape(1, C)

    deg = _degree_kernel(fwd_si, ones_1, zeros_1)
    dis3 = _dis_call(deg.reshape(NSC, NBLK, BLK)).reshape(NBLK, 1, BLK)

    cnt = _cnt_call(batch3)
    u = _u_call(batch3, dis3, cnt)
    t = _edge_scatter_h(u, rev_gi, rev_si, zeros_h)
    qt = _qt_call(t, u, dis3)

    g = _prep_call(x_p, dis3, W1)
    Ws = [W2, W3, W4, W5]
    bs = [b1r, b2r, b3r, b4r]
    for l in range(4):
        s = _edge_scatter_h(g, fwd_gi, fwd_si, zeros_h)
        g, _h = _layer_call(s, g, dis3, Ws[l], bs[l])
    s = _edge_scatter_h(g, fwd_gi, fwd_si, zeros_h)
    h5 = _comb5_call(s, g, dis3, b5r)

    return _final_call(qt, h5, W6, Wlin, b6r, blinr)


# speedup vs baseline: 2.1451x; 1.0643x over previous
import functools

import jax
import jax.numpy as jnp
from jax import lax
from jax.experimental import pallas as pl
from jax.experimental.pallas import tpu as pltpu
from jax.experimental.pallas import tpu_sc as plsc

N = 10000
E = 320000
D = 128
H = 128
C = 40
G = 64

NSC = 2
NTILE = 16
NW = NSC * NTILE
CHUNK = 128
BLK = 128
NBLK = 79
NROWP = NBLK * BLK
RPT = NROWP // NTILE
NCHA = 95
NCHB = 62
NCH_MAX = NCHA
PCH = NCH_MAX // 2
EPAD = NTILE * (NCHA + NCHB) * CHUNK
NEA = NTILE * NCHA * CHUNK
TRASH = N

_mesh = plsc.VectorSubcoreMesh(core_axis_name="c", subcore_axis_name="s")


def _edge_scatter_kernel(width):
    @functools.partial(
        pl.kernel,
        out_type=jax.ShapeDtypeStruct((NSC, NROWP, width), jnp.float32),
        mesh=_mesh,
        scratch_types=[
            pltpu.VMEM((NCH_MAX, CHUNK), jnp.int32),
            pltpu.VMEM((NCH_MAX, CHUNK), jnp.int32),
            pltpu.VMEM((CHUNK, width), jnp.float32),
            pltpu.VMEM_SHARED((NROWP, width), jnp.float32),
            pltpu.SemaphoreType.DMA,
        ],
    )
    def k(g_hbm, gi_hbm, si_hbm, z_hbm, out_hbm, gi_v, si_v, rows_v, acc_sh, sem):
        c = lax.axis_index("c")
        s = lax.axis_index("s")
        wid = c * NTILE + s
        base = s * RPT
        nch = lax.select(c == 0, NCHA, NCHB)
        pltpu.sync_copy(z_hbm, rows_v)
        for kk in range(4):
            pltpu.sync_copy(rows_v, acc_sh.at[pl.ds(base + kk * CHUNK, CHUNK)])
        pltpu.sync_copy(rows_v.at[pl.ds(0, RPT - 4 * CHUNK)],
                        acc_sh.at[pl.ds(base + 4 * CHUNK, RPT - 4 * CHUNK)])
        pltpu.sync_copy(gi_hbm.at[wid], gi_v)
        pltpu.sync_copy(si_hbm.at[wid], si_v)
        plsc.subcore_barrier()

        def body(ch, carry):
            pltpu.async_copy(g_hbm.at[gi_v.at[ch]], rows_v, sem).wait()
            pltpu.sync_copy(rows_v, acc_sh.at[si_v.at[ch]], add=True)
            return carry

        lax.fori_loop(0, nch, body, 0)
        plsc.subcore_barrier()
        for kk in range(4):
            pltpu.sync_copy(acc_sh.at[pl.ds(base + kk * CHUNK, CHUNK)], rows_v)
            pltpu.sync_copy(rows_v, out_hbm.at[c, pl.ds(base + kk * CHUNK, CHUNK)])
        tail = RPT - 4 * CHUNK
        pltpu.sync_copy(acc_sh.at[pl.ds(base + 4 * CHUNK, tail)],
                        rows_v.at[pl.ds(0, tail)])
        pltpu.sync_copy(rows_v.at[pl.ds(0, tail)],
                        out_hbm.at[c, pl.ds(base + 4 * CHUNK, tail)])

    return k


_edge_scatter_h = _edge_scatter_kernel(H)


@functools.partial(
    pl.kernel,
    out_type=jax.ShapeDtypeStruct((NSC * NROWP,), jnp.float32),
    mesh=_mesh,
    scratch_types=[
        pltpu.VMEM((NCH_MAX, CHUNK), jnp.int32),
        pltpu.VMEM((CHUNK,), jnp.float32),
        pltpu.VMEM((RPT,), jnp.float32),
        pltpu.VMEM_SHARED((NROWP,), jnp.float32),
        pltpu.SemaphoreType.DMA,
    ],
)
def _degree_kernel(si_hbm, ones_hbm, z_hbm, out_hbm, si_v, ones_v, zb_v, acc_sh, sem):
    c = lax.axis_index("c")
    s = lax.axis_index("s")
    wid = c * NTILE + s
    base = s * RPT
    nch = lax.select(c == 0, NCHA, NCHB)
    pltpu.sync_copy(z_hbm, zb_v)
    pltpu.sync_copy(zb_v, acc_sh.at[pl.ds(base, RPT)])
    pltpu.sync_copy(ones_hbm, ones_v)
    pltpu.sync_copy(si_hbm.at[wid], si_v)
    plsc.subcore_barrier()

    def body(ch, carry):
        pltpu.sync_copy(ones_v, acc_sh.at[si_v.at[ch]], add=True)
        return carry

    lax.fori_loop(0, nch, body, 0)
    plsc.subcore_barrier()
    pltpu.sync_copy(acc_sh.at[pl.ds(base, RPT)], zb_v)
    pltpu.sync_copy(zb_v, out_hbm.at[pl.ds(c * NROWP + base, RPT)])


def _dis_body(deg_ref, out_ref):
    d = deg_ref[...]
    out_ref[...] = lax.rsqrt(d[0] + d[1] + 1.0)


_dis_call = pl.pallas_call(
    _dis_body,
    out_shape=jax.ShapeDtypeStruct((NBLK, BLK), jnp.float32),
)

TBLK = 632
TNB = NROWP // TBLK


def _prep_body(x_ref, dis_ref, w_ref, o_ref):
    d = dis_ref[0, 0, :]
    o_ref[...] = (x_ref[...] @ w_ref[...]) * d[:, None]


_prep_call = pl.pallas_call(
    _prep_body,
    grid=(NBLK,),
    in_specs=[
        pl.BlockSpec((BLK, D), lambda i: (i, 0)),
        pl.BlockSpec((1, 1, BLK), lambda i: (i, 0, 0)),
        pl.BlockSpec((D, H), lambda i: (0, 0)),
    ],
    out_specs=pl.BlockSpec((BLK, H), lambda i: (i, 0)),
    out_shape=jax.ShapeDtypeStruct((NROWP, H), jnp.float32),
)


def _layer_body(s_ref, g_ref, dis_ref, w_ref, b_ref, gn_ref, h_ref):
    d = dis_ref[...]
    sp = s_ref[...]
    h = jnp.maximum((sp[0] + sp[1] + g_ref[...]) * d + b_ref[...], 0.0)
    h_ref[...] = h
    gn_ref[...] = (h @ w_ref[...]) * d


_layer_call = pl.pallas_call(
    _layer_body,
    grid=(TNB,),
    in_specs=[
        pl.BlockSpec((NSC, TBLK, H), lambda i: (0, i, 0)),
        pl.BlockSpec((TBLK, H), lambda i: (i, 0)),
        pl.BlockSpec((TBLK, 1), lambda i: (i, 0)),
        pl.BlockSpec((H, H), lambda i: (0, 0)),
        pl.BlockSpec((1, H), lambda i: (0, 0)),
    ],
    out_specs=[
        pl.BlockSpec((TBLK, H), lambda i: (i, 0)),
        pl.BlockSpec((TBLK, H), lambda i: (i, 0)),
    ],
    out_shape=[
        jax.ShapeDtypeStruct((NROWP, H), jnp.float32),
        jax.ShapeDtypeStruct((NROWP, H), jnp.float32),
    ],
)


def _comb5_body(s_ref, g_ref, dis_ref, b_ref, h_ref):
    d = dis_ref[0, 0, :]
    sp = s_ref[...]
    h_ref[...] = jnp.maximum((sp[0] + sp[1] + g_ref[...]) * d[:, None] + b_ref[...], 0.0)


_comb5_call = pl.pallas_call(
    _comb5_body,
    grid=(NBLK,),
    in_specs=[
        pl.BlockSpec((NSC, BLK, H), lambda i: (0, i, 0)),
        pl.BlockSpec((BLK, H), lambda i: (i, 0)),
        pl.BlockSpec((1, 1, BLK), lambda i: (i, 0, 0)),
        pl.BlockSpec((1, H), lambda i: (0, 0)),
    ],
    out_specs=pl.BlockSpec((BLK, H), lambda i: (i, 0)),
    out_shape=jax.ShapeDtypeStruct((NROWP, H), jnp.float32),
)


def _cnt_body(b_ref, cnt_ref):
    @pl.when(pl.program_id(0) == 0)
    def _init():
        cnt_ref[...] = jnp.zeros_like(cnt_ref)

    bb = b_ref[0, 0, :]
    oh = (bb[:, None] == lax.broadcasted_iota(jnp.int32, (BLK, G), 1)).astype(jnp.float32)
    cnt_ref[...] += jnp.sum(oh, axis=0, keepdims=True)


_cnt_call = pl.pallas_call(
    _cnt_body,
    grid=(NBLK,),
    in_specs=[pl.BlockSpec((1, 1, BLK), lambda i: (i, 0, 0))],
    out_specs=pl.BlockSpec((1, G), lambda i: (0, 0)),
    out_shape=jax.ShapeDtypeStruct((1, G), jnp.float32),
)


def _u_body(b_ref, dis_ref, cnt_ref, u_ref):
    bb = b_ref[0, 0, :]
    d = dis_ref[0, 0, :]
    oh = (bb[:, None] == lax.broadcasted_iota(jnp.int32, (BLK, H), 1)).astype(jnp.float32)
    cinv = 1.0 / jnp.maximum(cnt_ref[...], 1.0)
    cinv_p = jnp.concatenate([cinv, jnp.ones((1, H - G), jnp.float32)], axis=1)
    u_ref[...] = oh * d[:, None] * cinv_p


_u_call = pl.pallas_call(
    _u_body,
    grid=(NBLK,),
    in_specs=[
        pl.BlockSpec((1, 1, BLK), lambda i: (i, 0, 0)),
        pl.BlockSpec((1, 1, BLK), lambda i: (i, 0, 0)),
        pl.BlockSpec((1, G), lambda i: (0, 0)),
    ],
    out_specs=pl.BlockSpec((BLK, H), lambda i: (i, 0)),
    out_shape=jax.ShapeDtypeStruct((NROWP, H), jnp.float32),
)


def _qt_body(t_ref, u_ref, dis_ref, qt_ref):
    d = dis_ref[0, 0, :]
    tp = t_ref[...]
    rid = pl.program_id(0) * BLK + lax.broadcasted_iota(jnp.int32, (BLK, G), 0)
    qt = ((tp[0] + tp[1] + u_ref[...]) * d[:, None])[:, :G]
    qt_ref[...] = jnp.where(rid < N, qt, 0.0)


_qt_call = pl.pallas_call(
    _qt_body,
    grid=(NBLK,),
    in_specs=[
        pl.BlockSpec((NSC, BLK, H), lambda i: (0, i, 0)),
        pl.BlockSpec((BLK, H), lambda i: (i, 0)),
        pl.BlockSpec((1, 1, BLK), lambda i: (i, 0, 0)),
    ],
    out_specs=pl.BlockSpec((BLK, G), lambda i: (i, 0)),
    out_shape=jax.ShapeDtypeStruct((NROWP, G), jnp.float32),
)


def _final_body(qt_ref, h_ref, w6_ref, wlin_ref, b6_ref, blin_ref, o_ref):
    pooled = lax.dot_general(qt_ref[...], h_ref[...], (((0,), (0,)), ((), ())),
                             preferred_element_type=jnp.float32)
    wc = w6_ref[...] @ wlin_ref[...]
    o_ref[...] = pooled @ wc + b6_ref[...] @ wlin_ref[...] + blin_ref[...]


_final_call = pl.pallas_call(
    _final_body,
    out_shape=jax.ShapeDtypeStruct((G, C), jnp.float32),
)


def kernel(x, edge_index, batch, W1, b1, W2, b2, W3, b3, W4, b4, W5, b5, W6, b6,
           Wlin, blin):
    f32 = jnp.float32
    src = edge_index[0]
    dst = edge_index[1]
    pad = EPAD - E

    def slab(idx, fill):
        idx_p = jnp.concatenate([idx, jnp.full((pad,), fill, jnp.int32)])
        part_a = idx_p[:NEA].reshape(NTILE, NCHA, CHUNK)
        part_a = jnp.pad(part_a, ((0, 0), (0, NCH_MAX - NCHA), (0, 0)),
                         constant_values=fill)
        part_b = idx_p[NEA:].reshape(NTILE, NCHB, CHUNK)
        part_b = jnp.pad(part_b, ((0, 0), (0, NCH_MAX - NCHB), (0, 0)),
                         constant_values=fill)
        return jnp.concatenate([part_a, part_b], axis=0)

    fwd_gi = slab(src, 0)
    fwd_si = slab(dst, TRASH)
    rev_gi = slab(dst, 0)
    rev_si = slab(src, TRASH)

    zeros_h = jnp.zeros((CHUNK, H), f32)
    zeros_1 = jnp.zeros((RPT,), f32)
    ones_1 = jnp.ones((CHUNK,), f32)

    x_p = jnp.pad(x, ((0, NROWP - N), (0, 0)))
    batch3 = jnp.concatenate(
        [batch, jnp.full((NROWP - N,), G, jnp.int32)]).reshape(NBLK, 1, BLK)
    b1r = b1.reshape(1, H); b2r = b2.reshape(1, H); b3r = b3.reshape(1, H)
    b4r = b4.reshape(1, H); b5r = b5.reshape(1, H); b6r = b6.reshape(1, H)
    blinr = blin.reshape(1, C)

    deg = _degree_kernel(fwd_si, ones_1, zeros_1)
    dis2d = _dis_call(deg.reshape(NSC, NBLK, BLK))
    dis3 = dis2d.reshape(NBLK, 1, BLK)
    dis_col = dis2d.reshape(NROWP, 1)

    cnt = _cnt_call(batch3)
    u = _u_call(batch3, dis3, cnt)
    t = _edge_scatter_h(u, rev_gi, rev_si, zeros_h)
    qt = _qt_call(t, u, dis3)

    g = _prep_call(x_p, dis3, W1)
    Ws = [W2, W3, W4, W5]
    bs = [b1r, b2r, b3r, b4r]
    for l in range(4):
        s = _edge_scatter_h(g, fwd_gi, fwd_si, zeros_h)
        g, _h = _layer_call(s, g, dis_col, Ws[l], bs[l])
    s = _edge_scatter_h(g, fwd_gi, fwd_si, zeros_h)
    h5 = _comb5_call(s, g, dis3, b5r)

    return _final_call(qt, h5, W6, Wlin, b6r, blinr)
